# baseline (speedup 1.0000x reference)
import functools

import jax
import jax.numpy as jnp
from jax.experimental import pallas as pl
from jax.experimental.pallas import tpu as pltpu


def _se_kernel(x_ref, w1_ref, w2_ref, o_ref, g_ref, *, inv_hw, hh, ks):
    k = pl.program_id(1)

    @pl.when(k == 0)
    def _gates():
        x = x_ref[...].astype(jnp.float32)
        pooled = jnp.sum(x, axis=1) * inv_hw
        y1 = jnp.dot(pooled, w1_ref[...], preferred_element_type=jnp.float32)
        y1 = jnp.maximum(y1, 0.0)
        g_ref[...] = jax.nn.sigmoid(
            jnp.dot(y1, w2_ref[...], preferred_element_type=jnp.float32))

    g = g_ref[...][:, None, :]
    for j in range(ks):
        @pl.when(k == j)
        def _scale(j=j):
            xs = x_ref[:, j * hh:(j + 1) * hh, :].astype(jnp.float32)
            o_ref[...] = (xs * g).astype(o_ref.dtype)


def _pick_bt(b):
    for cand in (64, 32, 16, 8, 4, 2, 1):
        if b % cand == 0 and b // cand >= 2:
            return cand
    return b


@jax.jit
def _se_block(x_nchw, w1, w2):
    B, C, H, W = x_nchw.shape
    HW = H * W
    Cr = w1.shape[0]

    w1t = jnp.transpose(w1).astype(jnp.float32)
    w2t = jnp.transpose(w2).astype(jnp.float32)

    x_v = jnp.transpose(x_nchw, (0, 2, 3, 1)).reshape(B, HW, C)

    bt = _pick_bt(B)
    nb = B // bt

    cost = pl.CostEstimate(
        flops=int(3 * B * C * HW + 4 * B * C * Cr),
        transcendentals=int(B * C),
        bytes_accessed=int(2 * B * C * HW * x_nchw.dtype.itemsize
                           + 2 * C * Cr * 4),
    )

    ks = 2 if HW % 16 == 0 else 1
    hh = HW // ks
    out = pl.pallas_call(
        functools.partial(_se_kernel, inv_hw=1.0 / HW, hh=hh, ks=ks),
        out_shape=jax.ShapeDtypeStruct((B, HW, C), x_nchw.dtype),
        grid=(nb, ks),
        in_specs=[
            pl.BlockSpec((bt, HW, C), lambda b, k: (b, 0, 0)),
            pl.BlockSpec((C, Cr), lambda b, k: (0, 0)),
            pl.BlockSpec((Cr, C), lambda b, k: (0, 0)),
        ],
        out_specs=pl.BlockSpec((bt, hh, C), lambda b, k: (b, k, 0)),
        scratch_shapes=[pltpu.VMEM((bt, C), jnp.float32)],
        compiler_params=pltpu.CompilerParams(
            dimension_semantics=("parallel", "arbitrary")),
        cost_estimate=cost,
    )(x_v, w1t, w2t)

    return jnp.transpose(out.reshape(B, H, W, C), (0, 3, 1, 2))


def kernel(x_nchw, w1, w2):
    return _se_block(x_nchw, w1, w2)


# speedup vs baseline: 1.3703x; 1.3703x over previous
import functools

import jax
import jax.numpy as jnp
from jax.experimental import pallas as pl
from jax.experimental.pallas import tpu as pltpu


def _se_kernel(x_ref, w1_ref, w2_ref, o_ref, *, inv_hw):
    x = x_ref[...].astype(jnp.float32)
    pooled = jnp.sum(x, axis=1) * inv_hw
    y1 = jnp.dot(pooled, w1_ref[...], preferred_element_type=jnp.float32)
    y1 = jnp.maximum(y1, 0.0)
    g = jax.nn.sigmoid(
        jnp.dot(y1, w2_ref[...], preferred_element_type=jnp.float32))
    o_ref[...] = (x * g[:, None, :]).astype(o_ref.dtype)


def _pick_bt(b):
    for cand in (64, 32, 16, 8, 4, 2, 1):
        if b % cand == 0 and b // cand >= 2:
            return cand
    return b


@jax.jit
def _se_block(x_nchw, w1, w2):
    B, C, H, W = x_nchw.shape
    HW = H * W
    Cr = w1.shape[0]

    w1t = jnp.transpose(w1).astype(jnp.float32)
    w2t = jnp.transpose(w2).astype(jnp.float32)

    x_v = jnp.transpose(x_nchw, (0, 2, 3, 1)).reshape(B, HW, C)

    bt = _pick_bt(B)
    nb = B // bt

    cost = pl.CostEstimate(
        flops=int(3 * B * C * HW + 4 * B * C * Cr),
        transcendentals=int(B * C),
        bytes_accessed=int(2 * B * C * HW * x_nchw.dtype.itemsize
                           + 2 * C * Cr * 4),
    )

    out = pl.pallas_call(
        functools.partial(_se_kernel, inv_hw=1.0 / HW),
        out_shape=jax.ShapeDtypeStruct((B, HW, C), x_nchw.dtype),
        grid=(nb,),
        in_specs=[
            pl.BlockSpec((bt, HW, C), lambda b: (b, 0, 0)),
            pl.BlockSpec((C, Cr), lambda b: (0, 0)),
            pl.BlockSpec((Cr, C), lambda b: (0, 0)),
        ],
        out_specs=pl.BlockSpec((bt, HW, C), lambda b: (b, 0, 0)),
        compiler_params=pltpu.CompilerParams(
            dimension_semantics=("parallel",)),
        cost_estimate=cost,
    )(x_v, w1t, w2t)

    return jnp.transpose(out.reshape(B, H, W, C), (0, 3, 1, 2))


def kernel(x_nchw, w1, w2):
    return _se_block(x_nchw, w1, w2)
